# baseline (speedup 1.0000x reference)
import jax
import jax.numpy as jnp
from jax import lax
from jax.experimental import pallas as pl
from jax.experimental.pallas import tpu as pltpu
from jax.experimental.pallas import tpu_sc as plsc

N = 10000
E = 320000
D = 128
U = 32
H = 4
OUT = 128

NP = 10240
NC = 2
NS = 16
NW = NC * NS
CH = 80
NCHUNK = 128
CHS = 1024
NCHS = 10
EPW = CH * NCHUNK
EP = EPW * NW
TAB_W = 2 * H
RPT = NP // NS


_RB = 1024
_GRID = NP // _RB


def _row_spec(w):
    return pl.BlockSpec((_RB, w), lambda i: (i, 0))


def _full_spec(shape):
    return pl.BlockSpec(shape, lambda i: tuple(0 for _ in shape))


def _pre_body(ns_ref, wp_ref, bp_ref, wc_ref, pt_ref, x0_ref, xt_ref, tab_ref):
    x = jnp.maximum(
        jnp.dot(ns_ref[...], wp_ref[...], preferred_element_type=jnp.float32)
        + bp_ref[...], 0.0)
    x0_ref[...] = x
    xt = jnp.dot(x, wc_ref[...], preferred_element_type=jnp.float32)
    xt_ref[...] = xt
    tab_ref[...] = jnp.dot(xt, pt_ref[...], preferred_element_type=jnp.float32)


def _tc_pre(ns_p, W_pre, b_pre2, Wcat, Ptab):
    return pl.pallas_call(
        _pre_body,
        grid=(_GRID,),
        in_specs=[_row_spec(D), _full_spec((D, D)), _full_spec((1, D)),
                  _full_spec((D, D)), _full_spec((D, TAB_W))],
        out_specs=[_row_spec(D), _row_spec(D), _row_spec(TAB_W)],
        out_shape=[jax.ShapeDtypeStruct((NP, D), jnp.float32),
                   jax.ShapeDtypeStruct((NP, D), jnp.float32),
                   jax.ShapeDtypeStruct((NP, TAB_W), jnp.float32)],
    )(ns_p, W_pre, b_pre2, Wcat, Ptab)


def _combine(num_ref, den_ref, xp_ref, s_ref):
    num = num_ref[0] + num_ref[1]
    den4 = jnp.sum(den_ref[...], axis=0)
    den = jnp.dot(den4, s_ref[...], preferred_element_type=jnp.float32)
    att = num / jnp.maximum(den, 1e-20)
    return jnp.maximum(att, 0.0) + xp_ref[...]


def _mid_body(num_ref, den_ref, xp_ref, s_ref, wc_ref, pt_ref,
              x_ref, xt_ref, tab_ref):
    x = _combine(num_ref, den_ref, xp_ref, s_ref)
    x_ref[...] = x
    xt = jnp.dot(x, wc_ref[...], preferred_element_type=jnp.float32)
    xt_ref[...] = xt
    tab_ref[...] = jnp.dot(xt, pt_ref[...], preferred_element_type=jnp.float32)


def _tc_mid(num, den, x_prev, S4, Wcat, Ptab):
    return pl.pallas_call(
        _mid_body,
        grid=(_GRID,),
        in_specs=[pl.BlockSpec((NC, _RB, D), lambda i: (0, i, 0)),
                  pl.BlockSpec((NW, _RB, H), lambda i: (0, i, 0)),
                  _row_spec(D), _full_spec((H, D)), _full_spec((D, D)),
                  _full_spec((D, TAB_W))],
        out_specs=[_row_spec(D), _row_spec(D), _row_spec(TAB_W)],
        out_shape=[jax.ShapeDtypeStruct((NP, D), jnp.float32),
                   jax.ShapeDtypeStruct((NP, D), jnp.float32),
                   jax.ShapeDtypeStruct((NP, TAB_W), jnp.float32)],
    )(num, den, x_prev, S4, Wcat, Ptab)


def _fin_body(num_ref, den_ref, xp_ref, s_ref, wo_ref, bo_ref, out_ref):
    x = _combine(num_ref, den_ref, xp_ref, s_ref)
    out_ref[...] = (
        jnp.dot(x, wo_ref[...], preferred_element_type=jnp.float32) + bo_ref[...])


def _tc_fin(num, den, x_prev, S4, W_out, b_out2):
    return pl.pallas_call(
        _fin_body,
        grid=(_GRID,),
        in_specs=[pl.BlockSpec((NC, _RB, D), lambda i: (0, i, 0)),
                  pl.BlockSpec((NW, _RB, H), lambda i: (0, i, 0)),
                  _row_spec(D), _full_spec((H, D)), _full_spec((D, OUT)),
                  _full_spec((1, OUT))],
        out_specs=_row_spec(OUT),
        out_shape=jax.ShapeDtypeStruct((NP, OUT), jnp.float32),
    )(num, den, x_prev, S4, W_out, b_out2)


def _sc_score_body(tab_hbm, dst_hbm, src_hbm, w_hbm, den_hbm,
                   tab_v, den_v, didx_v, sidx_v, wbuf_v):
    c = lax.axis_index("c")
    s = lax.axis_index("s")
    wid = c * NS + s
    lane = lax.iota(jnp.int32, 16)
    zero16 = jnp.zeros((16,), jnp.float32)

    def _zden(r, carry):
        den_v[pl.ds(r * 16, 16)] = zero16
        return carry

    lax.fori_loop(0, NP * H // 16, _zden, 0)

    pltpu.sync_copy(tab_hbm, tab_v)

    base = wid * EPW

    def _chunk(i, carry):
        off = base + i * CHS
        pltpu.sync_copy(dst_hbm.at[pl.ds(off, CHS)], didx_v)
        pltpu.sync_copy(src_hbm.at[pl.ds(off, CHS)], sidx_v)

        @plsc.parallel_loop(0, CHS // 16, unroll=4)
        def _group(g):
            d16 = didx_v[pl.ds(g * 16, 16)]
            s16 = sidx_v[pl.ds(g * 16, 16)] * TAB_W
            d16t = d16 * TAB_W
            widx = (lane + g * 16) * H
            for h in range(H):
                ad = plsc.load_gather(tab_v, [d16t + h])
                asrc = plsc.load_gather(tab_v, [s16 + (H + h)])
                sc = ad + asrc
                sc = jnp.where(sc < 0.0, sc * 0.2, sc)
                sc = jnp.clip(sc, -2.0, 2.0)
                w = jnp.exp(sc)
                plsc.store_scatter(wbuf_v, [widx + h], w)

        def _dgroup(g2, carry2):
            for u in range(2):
                g = g2 * 2 + u
                d16h = didx_v[pl.ds(g * 16, 16)] * H
                widx = (lane + g * 16) * H
                for h in range(H):
                    w16 = plsc.load_gather(wbuf_v, [widx + h])
                    plsc.addupdate_scatter(den_v, [d16h + h], w16)
            return carry2

        lax.fori_loop(0, CHS // 32, _dgroup, 0)
        pltpu.sync_copy(wbuf_v.at[pl.ds(0, CHS * H)],
                        w_hbm.at[pl.ds(off * H, CHS * H)])
        return carry

    lax.fori_loop(0, NCHS, _chunk, 0)

    pltpu.sync_copy(den_v, den_hbm.at[wid])


_sc_score = pl.kernel(
    _sc_score_body,
    out_type=[jax.ShapeDtypeStruct((EP * H,), jnp.float32),
              jax.ShapeDtypeStruct((NW, NP * H), jnp.float32)],
    mesh=plsc.VectorSubcoreMesh(core_axis_name="c", subcore_axis_name="s"),
    compiler_params=pltpu.CompilerParams(needs_layout_passes=False),
    scratch_types=[
        pltpu.VMEM((NP * TAB_W,), jnp.float32),
        pltpu.VMEM((NP * H,), jnp.float32),
        pltpu.VMEM((CHS,), jnp.int32),
        pltpu.VMEM((CHS,), jnp.int32),
        pltpu.VMEM((CHS * H,), jnp.float32),
    ],
)


def _sc_accum_body(xt_hbm, w_hbm, dst_hbm, src_hbm, num_hbm,
                   didx0, didx1, didx2, didx3, sidx0, sidx1, sidx2, sidx3,
                   rows0, rows1, rows2, rows3, wbuf0, wbuf1, wbuf2, wbuf3,
                   accum_num,
                   gx0, gx1, gx2, gx3, gi0, gi1, gi2, gi3,
                   sn0, sn1, sn2, sn3):
    c = lax.axis_index("c")
    s = lax.axis_index("s")
    wid = c * NS + s
    lane = lax.iota(jnp.int32, 16)
    zero16 = jnp.zeros((16,), jnp.float32)

    bufs = ((didx0, sidx0, rows0, wbuf0, gx0, gi0, sn0),
            (didx1, sidx1, rows1, wbuf1, gx1, gi1, sn1),
            (didx2, sidx2, rows2, wbuf2, gx2, gi2, sn2),
            (didx3, sidx3, rows3, wbuf3, gx3, gi3, sn3))

    def _zrow(r, carry):
        for k in range(D // 16):
            rows0[r, pl.ds(k * 16, 16)] = zero16
        return carry

    lax.fori_loop(0, CH, _zrow, 0)

    for i in range(RPT // CH):
        pltpu.sync_copy(rows0, accum_num.at[pl.ds(s * RPT + i * CH, CH)])

    plsc.subcore_barrier()

    base = wid * EPW

    def _load(i, b, wait_prev):
        didx, sidx, rows, wbuf, gx, gi, sn = bufs[b]
        if wait_prev:
            pltpu.make_async_copy(rows, accum_num.at[didx], sn).wait()
        off = base + i * CH
        pltpu.sync_copy(src_hbm.at[pl.ds(off, CH)], sidx)
        pltpu.async_copy(xt_hbm.at[sidx], rows, gx)
        pltpu.async_copy(dst_hbm.at[pl.ds(off, CH)], didx, gi)
        pltpu.async_copy(w_hbm.at[pl.ds(off * H, CH * H)],
                         wbuf.at[pl.ds(0, CH * H)], gi)

    def _process(b):
        didx, sidx, rows, wbuf, gx, gi, sn = bufs[b]
        pltpu.make_async_copy(dst_hbm.at[pl.ds(0, CH)], didx, gi).wait()
        pltpu.make_async_copy(w_hbm.at[pl.ds(0, CH * H)],
                              wbuf.at[pl.ds(0, CH * H)], gi).wait()
        pltpu.make_async_copy(xt_hbm.at[sidx], rows, gx).wait()

        @plsc.parallel_loop(0, CH, unroll=4)
        def _srow(e):
            wvec = wbuf[pl.ds(e * H, 16)]
            for h in range(H):
                wsc = wvec[h]
                for k in range(U // 16):
                    col = h * U + k * 16
                    rows[e, pl.ds(col, 16)] = rows[e, pl.ds(col, 16)] * wsc

        pltpu.async_copy(rows, accum_num.at[didx], sn, add=True)

    for b in range(4):
        _load(b, b, False)

    def _quad(j, carry):
        _process(0)
        _process(1)
        _load(4 * j + 4, 0, True)
        _load(4 * j + 5, 1, True)
        _process(2)
        _process(3)
        _load(4 * j + 6, 2, True)
        _load(4 * j + 7, 3, True)
        return carry

    lax.fori_loop(0, NCHUNK // 4 - 1, _quad, 0)
    for b in range(4):
        _process(b)
    for b in range(4):
        didx, sidx, rows, wbuf, gx, gi, sn = bufs[b]
        pltpu.make_async_copy(rows, accum_num.at[didx], sn).wait()

    plsc.subcore_barrier()

    for i in range(RPT // CH):
        pltpu.sync_copy(accum_num.at[pl.ds(s * RPT + i * CH, CH)],
                        num_hbm.at[c, pl.ds(s * RPT + i * CH, CH)])


_sc_accum = pl.kernel(
    _sc_accum_body,
    out_type=jax.ShapeDtypeStruct((NC, NP, D), jnp.float32),
    mesh=plsc.VectorSubcoreMesh(core_axis_name="c", subcore_axis_name="s"),
    compiler_params=pltpu.CompilerParams(needs_layout_passes=False),
    scratch_types=(
        [pltpu.VMEM((CH,), jnp.int32)] * 8
        + [pltpu.VMEM((CH, D), jnp.float32)] * 4
        + [pltpu.VMEM((CH * H + 16,), jnp.float32)] * 4
        + [pltpu.VMEM_SHARED((NP, D), jnp.float32)]
        + [pltpu.SemaphoreType.DMA] * 12
    ),
)


def _make_ptab(A):
    A2 = A[:, :, 0]
    eye = jnp.eye(H, dtype=jnp.float32)
    Pd = jnp.einsum("hj,hk->hjk", A2[:, :U], eye).reshape(H * U, H)
    Ps = jnp.einsum("hj,hk->hjk", A2[:, U:], eye).reshape(H * U, H)
    return jnp.concatenate([Pd, Ps], axis=1)


def _gat_layer(xt, tab, dst_p, src_p):
    w, den = _sc_score(tab.reshape(-1), dst_p, src_p)
    num = _sc_accum(xt, w, dst_p, src_p)
    return num, den.reshape(NW, NP, H)


def kernel(node_states, edges, W_pre, b_pre, W_att1, A_att1, W_att2, A_att2,
           W_out, b_out):
    f32 = jnp.float32
    ns_p = jnp.zeros((NP, D), f32).at[:N].set(node_states.astype(f32))

    e32 = edges.astype(jnp.int32)
    padlen = EP - E
    padidx = N + jnp.arange(padlen, dtype=jnp.int32) % (NP - N)
    dst_p = jnp.concatenate([e32[:, 0], padidx])
    src_p = jnp.concatenate([e32[:, 1], padidx])

    W1cat = jnp.transpose(W_att1, (1, 0, 2)).reshape(D, H * U)
    W2cat = jnp.transpose(W_att2, (1, 0, 2)).reshape(D, H * U)
    Ptab1 = _make_ptab(A_att1)
    Ptab2 = _make_ptab(A_att2)
    b_pre2 = b_pre.reshape(1, D).astype(f32)
    b_out2 = b_out.reshape(1, OUT).astype(f32)

    row_i = lax.broadcasted_iota(jnp.int32, (H, D), 0)
    col_h = lax.broadcasted_iota(jnp.int32, (H, D), 1) // U
    S4 = (row_i == col_h).astype(f32)

    x0, xt1, tab1 = _tc_pre(ns_p, W_pre.astype(f32), b_pre2, W1cat, Ptab1)
    num1, den1 = _gat_layer(xt1, tab1, dst_p, src_p)
    x1, xt2, tab2 = _tc_mid(num1, den1, x0, S4, W2cat, Ptab2)
    num2, den2 = _gat_layer(xt2, tab2, dst_p, src_p)
    outp = _tc_fin(num2, den2, x1, S4, W_out.astype(f32), b_out2)
    return outp[:N]


# speedup vs baseline: 1.0069x; 1.0069x over previous
import jax
import jax.numpy as jnp
from jax import lax
from jax.experimental import pallas as pl
from jax.experimental.pallas import tpu as pltpu
from jax.experimental.pallas import tpu_sc as plsc

N = 10000
E = 320000
D = 128
U = 32
H = 4
OUT = 128

NP = 10240
NC = 2
NS = 16
NW = NC * NS
CH = 80
NCHUNK = 128
CHS = 1024
NCHS = 10
EPW = CH * NCHUNK
EP = EPW * NW
TAB_W = 2 * H
RPT = NP // NS


_RB = 1024
_GRID = NP // _RB


def _row_spec(w):
    return pl.BlockSpec((_RB, w), lambda i: (i, 0))


def _full_spec(shape):
    return pl.BlockSpec(shape, lambda i: tuple(0 for _ in shape))


def _pre_body(ns_ref, wp_ref, bp_ref, wc_ref, pt_ref, x0_ref, xt_ref, tab_ref):
    x = jnp.maximum(
        jnp.dot(ns_ref[...], wp_ref[...], preferred_element_type=jnp.float32)
        + bp_ref[...], 0.0)
    x0_ref[...] = x
    xt = jnp.dot(x, wc_ref[...], preferred_element_type=jnp.float32)
    xt_ref[...] = xt
    tab_ref[...] = jnp.dot(xt, pt_ref[...], preferred_element_type=jnp.float32)


def _tc_pre(ns_p, W_pre, b_pre2, Wcat, Ptab):
    return pl.pallas_call(
        _pre_body,
        grid=(_GRID,),
        in_specs=[_row_spec(D), _full_spec((D, D)), _full_spec((1, D)),
                  _full_spec((D, D)), _full_spec((D, TAB_W))],
        out_specs=[_row_spec(D), _row_spec(D), _row_spec(TAB_W)],
        out_shape=[jax.ShapeDtypeStruct((NP, D), jnp.float32),
                   jax.ShapeDtypeStruct((NP, D), jnp.float32),
                   jax.ShapeDtypeStruct((NP, TAB_W), jnp.float32)],
    )(ns_p, W_pre, b_pre2, Wcat, Ptab)


def _combine(num_ref, den_ref, xp_ref, s_ref):
    num = num_ref[0] + num_ref[1]
    den4 = jnp.sum(den_ref[...], axis=0)
    den = jnp.dot(den4, s_ref[...], preferred_element_type=jnp.float32)
    att = num / jnp.maximum(den, 1e-20)
    return jnp.maximum(att, 0.0) + xp_ref[...]


def _mid_body(num_ref, den_ref, xp_ref, s_ref, wc_ref, pt_ref,
              x_ref, xt_ref, tab_ref):
    x = _combine(num_ref, den_ref, xp_ref, s_ref)
    x_ref[...] = x
    xt = jnp.dot(x, wc_ref[...], preferred_element_type=jnp.float32)
    xt_ref[...] = xt
    tab_ref[...] = jnp.dot(xt, pt_ref[...], preferred_element_type=jnp.float32)


def _tc_mid(num, den, x_prev, S4, Wcat, Ptab):
    return pl.pallas_call(
        _mid_body,
        grid=(_GRID,),
        in_specs=[pl.BlockSpec((NC, _RB, D), lambda i: (0, i, 0)),
                  pl.BlockSpec((NW, _RB, H), lambda i: (0, i, 0)),
                  _row_spec(D), _full_spec((H, D)), _full_spec((D, D)),
                  _full_spec((D, TAB_W))],
        out_specs=[_row_spec(D), _row_spec(D), _row_spec(TAB_W)],
        out_shape=[jax.ShapeDtypeStruct((NP, D), jnp.float32),
                   jax.ShapeDtypeStruct((NP, D), jnp.float32),
                   jax.ShapeDtypeStruct((NP, TAB_W), jnp.float32)],
    )(num, den, x_prev, S4, Wcat, Ptab)


def _fin_body(num_ref, den_ref, xp_ref, s_ref, wo_ref, bo_ref, out_ref):
    x = _combine(num_ref, den_ref, xp_ref, s_ref)
    out_ref[...] = (
        jnp.dot(x, wo_ref[...], preferred_element_type=jnp.float32) + bo_ref[...])


def _tc_fin(num, den, x_prev, S4, W_out, b_out2):
    return pl.pallas_call(
        _fin_body,
        grid=(_GRID,),
        in_specs=[pl.BlockSpec((NC, _RB, D), lambda i: (0, i, 0)),
                  pl.BlockSpec((NW, _RB, H), lambda i: (0, i, 0)),
                  _row_spec(D), _full_spec((H, D)), _full_spec((D, OUT)),
                  _full_spec((1, OUT))],
        out_specs=_row_spec(OUT),
        out_shape=jax.ShapeDtypeStruct((NP, OUT), jnp.float32),
    )(num, den, x_prev, S4, W_out, b_out2)


def _sc_score_body(tab_hbm, dst_hbm, src_hbm, w_hbm, den_hbm,
                   tab_v, den_v, didx_v, sidx_v, wbuf_v):
    c = lax.axis_index("c")
    s = lax.axis_index("s")
    wid = c * NS + s
    lane = lax.iota(jnp.int32, 16)
    zero16 = jnp.zeros((16,), jnp.float32)

    def _zden(r, carry):
        den_v[pl.ds(r * 16, 16)] = zero16
        return carry

    lax.fori_loop(0, NP * H // 16, _zden, 0)

    pltpu.sync_copy(tab_hbm, tab_v)

    base = wid * EPW

    def _chunk(i, carry):
        off = base + i * CHS
        pltpu.sync_copy(dst_hbm.at[pl.ds(off, CHS)], didx_v)
        pltpu.sync_copy(src_hbm.at[pl.ds(off, CHS)], sidx_v)

        @plsc.parallel_loop(0, CHS // 16, unroll=2)
        def _group(g):
            d16 = didx_v[pl.ds(g * 16, 16)]
            s16 = sidx_v[pl.ds(g * 16, 16)] * TAB_W
            d16t = d16 * TAB_W
            widx = (lane + g * 16) * H
            for h in range(H):
                ad = plsc.load_gather(tab_v, [d16t + h])
                asrc = plsc.load_gather(tab_v, [s16 + (H + h)])
                sc = ad + asrc
                sc = jnp.where(sc < 0.0, sc * 0.2, sc)
                sc = jnp.clip(sc, -2.0, 2.0)
                w = jnp.exp(sc)
                plsc.store_scatter(wbuf_v, [widx + h], w)

        def _dgroup(g, carry2):
            d16h = didx_v[pl.ds(g * 16, 16)] * H
            widx = (lane + g * 16) * H
            for h in range(H):
                w16 = plsc.load_gather(wbuf_v, [widx + h])
                plsc.addupdate_scatter(den_v, [d16h + h], w16)
            return carry2

        lax.fori_loop(0, CHS // 16, _dgroup, 0)
        pltpu.sync_copy(wbuf_v.at[pl.ds(0, CHS * H)],
                        w_hbm.at[pl.ds(off * H, CHS * H)])
        return carry

    lax.fori_loop(0, NCHS, _chunk, 0)

    pltpu.sync_copy(den_v, den_hbm.at[wid])


_sc_score = pl.kernel(
    _sc_score_body,
    out_type=[jax.ShapeDtypeStruct((EP * H,), jnp.float32),
              jax.ShapeDtypeStruct((NW, NP * H), jnp.float32)],
    mesh=plsc.VectorSubcoreMesh(core_axis_name="c", subcore_axis_name="s"),
    compiler_params=pltpu.CompilerParams(needs_layout_passes=False),
    scratch_types=[
        pltpu.VMEM((NP * TAB_W,), jnp.float32),
        pltpu.VMEM((NP * H,), jnp.float32),
        pltpu.VMEM((CHS,), jnp.int32),
        pltpu.VMEM((CHS,), jnp.int32),
        pltpu.VMEM((CHS * H,), jnp.float32),
    ],
)


def _sc_accum_body(xt_hbm, w_hbm, dst_hbm, src_hbm, num_hbm,
                   didx0, didx1, didx2, didx3, sidx0, sidx1, sidx2, sidx3,
                   rows0, rows1, rows2, rows3, wbuf0, wbuf1, wbuf2, wbuf3,
                   accum_num,
                   gx0, gx1, gx2, gx3, gi0, gi1, gi2, gi3,
                   sn0, sn1, sn2, sn3):
    c = lax.axis_index("c")
    s = lax.axis_index("s")
    wid = c * NS + s
    lane = lax.iota(jnp.int32, 16)
    zero16 = jnp.zeros((16,), jnp.float32)

    bufs = ((didx0, sidx0, rows0, wbuf0, gx0, gi0, sn0),
            (didx1, sidx1, rows1, wbuf1, gx1, gi1, sn1),
            (didx2, sidx2, rows2, wbuf2, gx2, gi2, sn2),
            (didx3, sidx3, rows3, wbuf3, gx3, gi3, sn3))

    def _zrow(r, carry):
        for k in range(D // 16):
            rows0[r, pl.ds(k * 16, 16)] = zero16
        return carry

    lax.fori_loop(0, CH, _zrow, 0)

    for i in range(RPT // CH):
        pltpu.sync_copy(rows0, accum_num.at[pl.ds(s * RPT + i * CH, CH)])

    plsc.subcore_barrier()

    base = wid * EPW

    def _load(i, b, wait_prev):
        didx, sidx, rows, wbuf, gx, gi, sn = bufs[b]
        if wait_prev:
            pltpu.make_async_copy(rows, accum_num.at[didx], sn).wait()
        off = base + i * CH
        pltpu.sync_copy(src_hbm.at[pl.ds(off, CH)], sidx)
        pltpu.async_copy(xt_hbm.at[sidx], rows, gx)
        pltpu.async_copy(dst_hbm.at[pl.ds(off, CH)], didx, gi)
        pltpu.async_copy(w_hbm.at[pl.ds(off * H, CH * H)],
                         wbuf.at[pl.ds(0, CH * H)], gi)

    def _process(b):
        didx, sidx, rows, wbuf, gx, gi, sn = bufs[b]
        pltpu.make_async_copy(dst_hbm.at[pl.ds(0, CH)], didx, gi).wait()
        pltpu.make_async_copy(w_hbm.at[pl.ds(0, CH * H)],
                              wbuf.at[pl.ds(0, CH * H)], gi).wait()
        pltpu.make_async_copy(xt_hbm.at[sidx], rows, gx).wait()

        @plsc.parallel_loop(0, CH, unroll=2)
        def _srow(e):
            wvec = wbuf[pl.ds(e * H, 16)]
            for h in range(H):
                wsc = wvec[h]
                for k in range(U // 16):
                    col = h * U + k * 16
                    rows[e, pl.ds(col, 16)] = rows[e, pl.ds(col, 16)] * wsc

        pltpu.async_copy(rows, accum_num.at[didx], sn, add=True)

    for b in range(4):
        _load(b, b, False)

    def _quad(j, carry):
        _process(0)
        _process(1)
        _load(4 * j + 4, 0, True)
        _load(4 * j + 5, 1, True)
        _process(2)
        _process(3)
        _load(4 * j + 6, 2, True)
        _load(4 * j + 7, 3, True)
        return carry

    lax.fori_loop(0, NCHUNK // 4 - 1, _quad, 0)
    for b in range(4):
        _process(b)
    for b in range(4):
        didx, sidx, rows, wbuf, gx, gi, sn = bufs[b]
        pltpu.make_async_copy(rows, accum_num.at[didx], sn).wait()

    plsc.subcore_barrier()

    for i in range(RPT // CH):
        pltpu.sync_copy(accum_num.at[pl.ds(s * RPT + i * CH, CH)],
                        num_hbm.at[c, pl.ds(s * RPT + i * CH, CH)])


_sc_accum = pl.kernel(
    _sc_accum_body,
    out_type=jax.ShapeDtypeStruct((NC, NP, D), jnp.float32),
    mesh=plsc.VectorSubcoreMesh(core_axis_name="c", subcore_axis_name="s"),
    compiler_params=pltpu.CompilerParams(needs_layout_passes=False),
    scratch_types=(
        [pltpu.VMEM((CH,), jnp.int32)] * 8
        + [pltpu.VMEM((CH, D), jnp.float32)] * 4
        + [pltpu.VMEM((CH * H + 16,), jnp.float32)] * 4
        + [pltpu.VMEM_SHARED((NP, D), jnp.float32)]
        + [pltpu.SemaphoreType.DMA] * 12
    ),
)


def _make_ptab(A):
    A2 = A[:, :, 0]
    eye = jnp.eye(H, dtype=jnp.float32)
    Pd = jnp.einsum("hj,hk->hjk", A2[:, :U], eye).reshape(H * U, H)
    Ps = jnp.einsum("hj,hk->hjk", A2[:, U:], eye).reshape(H * U, H)
    return jnp.concatenate([Pd, Ps], axis=1)


def _gat_layer(xt, tab, dst_p, src_p):
    w, den = _sc_score(tab.reshape(-1), dst_p, src_p)
    num = _sc_accum(xt, w, dst_p, src_p)
    return num, den.reshape(NW, NP, H)


def kernel(node_states, edges, W_pre, b_pre, W_att1, A_att1, W_att2, A_att2,
           W_out, b_out):
    f32 = jnp.float32
    ns_p = jnp.zeros((NP, D), f32).at[:N].set(node_states.astype(f32))

    e32 = edges.astype(jnp.int32)
    padlen = EP - E
    padidx = N + jnp.arange(padlen, dtype=jnp.int32) % (NP - N)
    dst_p = jnp.concatenate([e32[:, 0], padidx])
    src_p = jnp.concatenate([e32[:, 1], padidx])

    W1cat = jnp.transpose(W_att1, (1, 0, 2)).reshape(D, H * U)
    W2cat = jnp.transpose(W_att2, (1, 0, 2)).reshape(D, H * U)
    Ptab1 = _make_ptab(A_att1)
    Ptab2 = _make_ptab(A_att2)
    b_pre2 = b_pre.reshape(1, D).astype(f32)
    b_out2 = b_out.reshape(1, OUT).astype(f32)

    row_i = lax.broadcasted_iota(jnp.int32, (H, D), 0)
    col_h = lax.broadcasted_iota(jnp.int32, (H, D), 1) // U
    S4 = (row_i == col_h).astype(f32)

    x0, xt1, tab1 = _tc_pre(ns_p, W_pre.astype(f32), b_pre2, W1cat, Ptab1)
    num1, den1 = _gat_layer(xt1, tab1, dst_p, src_p)
    x1, xt2, tab2 = _tc_mid(num1, den1, x0, S4, W2cat, Ptab2)
    num2, den2 = _gat_layer(xt2, tab2, dst_p, src_p)
    outp = _tc_fin(num2, den2, x1, S4, W_out.astype(f32), b_out2)
    return outp[:N]
